# baseline (speedup 1.0000x reference)
import jax
import jax.numpy as jnp
from jax.experimental import pallas as pl
from jax.experimental.pallas import tpu as pltpu

_N = 16384
_D = 1024
_CH = 4096
_NCHUNK = _N // _CH
_NBUF = 3


def _body(x_hbm, o_hbm, *rest):
    bufs = rest[:_NBUF]
    insem, outsem = rest[_NBUF], rest[_NBUF + 1]

    def in_copy(k):
        return pltpu.make_async_copy(
            x_hbm.at[pl.ds(k * _CH, _CH)], bufs[k % _NBUF], insem.at[k % _NBUF])

    half = _CH // 2

    def out_half(k, h):
        sl = pl.ds(h * half, half)
        return pltpu.make_async_copy(
            bufs[k % _NBUF].at[sl],
            o_hbm.at[pl.ds(k * _CH + h * half, half)],
            outsem.at[k % _NBUF, h])

    for k in range(_NBUF):
        in_copy(k).start()
    for k in range(_NCHUNK):
        in_copy(k).wait()
        buf = bufs[k % _NBUF]
        for h in range(2):
            sl = pl.ds(h * half, half)
            row_col = (jax.lax.broadcasted_iota(jnp.int32, (half, 1), 0)
                       + (k * _CH + h * half)).astype(jnp.float32)
            buf[sl, :] = buf[sl, :] * 2.0 + row_col
            out_half(k, h).start()
        if k + _NBUF < _NCHUNK:
            out_half(k, 0).wait()
            out_half(k, 1).wait()
            in_copy(k + _NBUF).start()
    for k in range(_NCHUNK - _NBUF, _NCHUNK):
        out_half(k, 0).wait()
        out_half(k, 1).wait()


def kernel(input_tensor):
    return pl.pallas_call(
        _body,
        in_specs=[pl.BlockSpec(memory_space=pl.ANY)],
        out_specs=pl.BlockSpec(memory_space=pl.ANY),
        out_shape=jax.ShapeDtypeStruct((_N, _D), input_tensor.dtype),
        scratch_shapes=(
            [pltpu.VMEM((_CH, _D), jnp.float32) for _ in range(_NBUF)]
            + [pltpu.SemaphoreType.DMA((_NBUF,)),
               pltpu.SemaphoreType.DMA((_NBUF, 2))]
        ),
        compiler_params=pltpu.CompilerParams(
            vmem_limit_bytes=64 * 1024 * 1024,
        ),
    )(input_tensor)


# speedup vs baseline: 1.0031x; 1.0031x over previous
import jax
import jax.numpy as jnp
from jax.experimental import pallas as pl
from jax.experimental.pallas import tpu as pltpu

_N = 16384
_D = 1024
_CH = 4096
_NCHUNK = _N // _CH
_NBUF = 3
_QUARTER = _CH // 4


def _body(x_hbm, o_hbm, *rest):
    bufs = rest[:_NBUF]
    insem, outsem = rest[_NBUF], rest[_NBUF + 1]
    qsem = rest[_NBUF + 2]

    def in_copy(k):
        return pltpu.make_async_copy(
            x_hbm.at[pl.ds(k * _CH, _CH)], bufs[k % _NBUF], insem.at[k % _NBUF])

    def out_copy(k):
        return pltpu.make_async_copy(
            bufs[k % _NBUF], o_hbm.at[pl.ds(k * _CH, _CH)], outsem.at[k % _NBUF])

    def scale(buf, start, length, base):
        sl = pl.ds(start, length)
        row_col = (jax.lax.broadcasted_iota(jnp.int32, (length, 1), 0)
                   + base).astype(jnp.float32)
        buf[sl, :] = buf[sl, :] * 2.0 + row_col

    for k in range(_NBUF):
        in_copy(k).start()
    for k in range(_NCHUNK - 1):
        in_copy(k).wait()
        scale(bufs[k % _NBUF], 0, _CH, k * _CH)
        out_copy(k).start()
        if k + _NBUF < _NCHUNK:
            out_copy(k).wait()
            in_copy(k + _NBUF).start()

    last = _NCHUNK - 1
    in_copy(last).wait()
    buf = bufs[last % _NBUF]
    for q in range(4):
        scale(buf, q * _QUARTER, _QUARTER, last * _CH + q * _QUARTER)
        pltpu.make_async_copy(
            buf.at[pl.ds(q * _QUARTER, _QUARTER)],
            o_hbm.at[pl.ds(last * _CH + q * _QUARTER, _QUARTER)],
            qsem.at[q]).start()

    for k in range(_NCHUNK - _NBUF, _NCHUNK - 1):
        out_copy(k).wait()
    for q in range(4):
        pltpu.make_async_copy(
            buf.at[pl.ds(q * _QUARTER, _QUARTER)],
            o_hbm.at[pl.ds(last * _CH + q * _QUARTER, _QUARTER)],
            qsem.at[q]).wait()


def kernel(input_tensor):
    return pl.pallas_call(
        _body,
        in_specs=[pl.BlockSpec(memory_space=pl.ANY)],
        out_specs=pl.BlockSpec(memory_space=pl.ANY),
        out_shape=jax.ShapeDtypeStruct((_N, _D), input_tensor.dtype),
        scratch_shapes=(
            [pltpu.VMEM((_CH, _D), jnp.float32) for _ in range(_NBUF)]
            + [pltpu.SemaphoreType.DMA((_NBUF,)),
               pltpu.SemaphoreType.DMA((_NBUF,)),
               pltpu.SemaphoreType.DMA((4,))]
        ),
        compiler_params=pltpu.CompilerParams(
            vmem_limit_bytes=64 * 1024 * 1024,
        ),
    )(input_tensor)


# speedup vs baseline: 1.0037x; 1.0005x over previous
import jax
import jax.numpy as jnp
from jax.experimental import pallas as pl
from jax.experimental.pallas import tpu as pltpu

_N = 16384
_D = 1024
_CH = 4096
_NCHUNK = _N // _CH
_NBUF = 3


def _body(x_hbm, o_hbm, *rest):
    bufs = rest[:_NBUF]
    insem, outsem = rest[_NBUF], rest[_NBUF + 1]

    def in_copy(k):
        return pltpu.make_async_copy(
            x_hbm.at[pl.ds(k * _CH, _CH)], bufs[k % _NBUF], insem.at[k % _NBUF])

    def out_copy(k):
        return pltpu.make_async_copy(
            bufs[k % _NBUF], o_hbm.at[pl.ds(k * _CH, _CH)], outsem.at[k % _NBUF])

    for k in range(_NBUF):
        in_copy(k).start()
    for k in range(_NCHUNK):
        in_copy(k).wait()
        buf = bufs[k % _NBUF]
        row_col = (jax.lax.broadcasted_iota(jnp.int32, (_CH, 1), 0)
                   + k * _CH).astype(jnp.float32)
        buf[...] = buf[...] * 2.0 + row_col
        out_copy(k).start()
        if k + _NBUF < _NCHUNK:
            out_copy(k).wait()
            in_copy(k + _NBUF).start()
    for k in range(_NCHUNK - _NBUF, _NCHUNK):
        out_copy(k).wait()


def kernel(input_tensor):
    return pl.pallas_call(
        _body,
        in_specs=[pl.BlockSpec(memory_space=pl.ANY)],
        out_specs=pl.BlockSpec(memory_space=pl.ANY),
        out_shape=jax.ShapeDtypeStruct((_N, _D), input_tensor.dtype),
        scratch_shapes=(
            [pltpu.VMEM((_CH, _D), jnp.float32) for _ in range(_NBUF)]
            + [pltpu.SemaphoreType.DMA((_NBUF,)),
               pltpu.SemaphoreType.DMA((_NBUF,))]
        ),
        compiler_params=pltpu.CompilerParams(
            vmem_limit_bytes=64 * 1024 * 1024,
        ),
    )(input_tensor)
